# baseline (speedup 1.0000x reference)
import functools

import jax
import jax.numpy as jnp
from jax import lax
from jax.experimental import pallas as pl
from jax.experimental.pallas import tpu as pltpu
from jax.experimental.pallas import tpu_sc as plsc

VOCAB = 100000
DIM1 = 64
DIM2 = 128
DIM = DIM1 + DIM2
BATCH = 4096
SEQ = 200
N = BATCH * SEQ

NUM_CORES = 2
NUM_SUBCORES = 16
NW = NUM_CORES * NUM_SUBCORES
PER_W = N // NW

CHUNK = 128
STEPS = PER_W // CHUNK
SUPER = 10
GROUPS = STEPS // SUPER
LANES = 16


def _make_kernel():
    mesh = plsc.VectorSubcoreMesh(core_axis_name="c", subcore_axis_name="s")

    @functools.partial(
        pl.kernel,
        mesh=mesh,
        out_type=jax.ShapeDtypeStruct((N, DIM), jnp.float32),
        scratch_types=[
            pltpu.VMEM((SUPER, CHUNK), jnp.int32),
            pltpu.VMEM((SUPER, CHUNK), jnp.int32),
            pltpu.VMEM((CHUNK, DIM), jnp.float32),
            pltpu.VMEM((CHUNK, DIM), jnp.float32),
            pltpu.VMEM((CHUNK, DIM2), jnp.float32),
            pltpu.VMEM((CHUNK, DIM2), jnp.float32),
            pltpu.SemaphoreType.DMA,
            pltpu.SemaphoreType.DMA,
            pltpu.SemaphoreType.DMA,
            pltpu.SemaphoreType.DMA,
        ],
    )
    def stack_embed(
        words_hbm,
        ta_hbm,
        t2_hbm,
        out_hbm,
        idx_a,
        idx_b,
        comb_a,
        comb_b,
        r2_a,
        r2_b,
        sem_ga,
        sem_gb,
        sem_ia,
        sem_ib,
    ):
        wid = lax.axis_index("s") * NUM_CORES + lax.axis_index("c")
        base = wid * PER_W
        idx = (idx_a, idx_b)
        comb = (comb_a, comb_b)
        r2 = (r2_a, r2_b)
        sem_g = (sem_ga, sem_gb)
        sem_i = (sem_ia, sem_ib)

        def issue(idx_row, b):
            pltpu.async_copy(
                ta_hbm.at[idx_row], comb[b].at[:, pl.ds(0, DIM2)], sem_g[b]
            )
            pltpu.async_copy(t2_hbm.at[idx_row], r2[b], sem_g[b])

        def drain(idx_row, b):
            pltpu.make_async_copy(
                ta_hbm.at[idx_row], comb[b].at[:, pl.ds(0, DIM2)], sem_g[b]
            ).wait()
            pltpu.make_async_copy(t2_hbm.at[idx_row], r2[b], sem_g[b]).wait()

        def process(j, b, cur_row, issue_next):
            drain(cur_row, b)
            issue_next()

            def tail2(jj, c):
                for r in range(2):
                    row = jj * 2 + r
                    for k in range(DIM1 // LANES):
                        comb[b][row, pl.ds(DIM2 + k * LANES, LANES)] = r2[b][
                            row, pl.ds(DIM1 + k * LANES, LANES)
                        ]
                return c

            lax.fori_loop(0, CHUNK // 2, tail2, 0)
            pltpu.sync_copy(comb[b], out_hbm.at[pl.ds(base + j * CHUNK, CHUNK)])

        def group(g, p, prefetch, t):
            prefetch()
            for c in range(SUPER):
                b = c % 2
                if c < SUPER - 1:
                    nxt = lambda c=c, b=b: issue(idx[p].at[c + 1], b ^ 1)
                elif p == 0:
                    def nxt(b=b):
                        pltpu.make_async_copy(
                            words_hbm.at[wid, 0], idx[1], sem_i[1]
                        ).wait()
                        issue(idx[1].at[0], b ^ 1)

                else:
                    def nxt(b=b):
                        @pl.when(t < GROUPS // 2 - 1)
                        def _():
                            pltpu.make_async_copy(
                                words_hbm.at[wid, 0], idx[0], sem_i[0]
                            ).wait()
                            issue(idx[0].at[0], b ^ 1)

                process(g * SUPER + c, b, idx[p].at[c], nxt)

        def body(t, carry):
            g0 = t * 2
            g1 = g0 + 1

            def pre0():
                pltpu.async_copy(words_hbm.at[wid, g1], idx[1], sem_i[1])

            def pre1():
                @pl.when(t < GROUPS // 2 - 1)
                def _():
                    pltpu.async_copy(words_hbm.at[wid, g0 + 2], idx[0], sem_i[0])

            group(g0, 0, pre0, t)
            group(g1, 1, pre1, t)
            return carry

        pltpu.sync_copy(words_hbm.at[wid, 0], idx[0])
        issue(idx[0].at[0], 0)
        lax.fori_loop(0, GROUPS // 2, body, 0)

    return stack_embed


_STACK_EMBED = _make_kernel()

_TC_BLOCK = 1000


def _concat_body(t1_ref, t2_ref, o_ref):
    o_ref[:, 0:DIM1] = t1_ref[...]
    o_ref[:, DIM1:DIM2] = t2_ref[:, 0:DIM1]


_CONCAT_TC = pl.pallas_call(
    _concat_body,
    grid=(VOCAB // _TC_BLOCK,),
    in_specs=[
        pl.BlockSpec((_TC_BLOCK, DIM1), lambda i: (i, 0)),
        pl.BlockSpec((_TC_BLOCK, DIM2), lambda i: (i, 0)),
    ],
    out_specs=pl.BlockSpec((_TC_BLOCK, DIM2), lambda i: (i, 0)),
    out_shape=jax.ShapeDtypeStruct((VOCAB, DIM2), jnp.float32),
)


def kernel(words, table1, table2):
    ta = _CONCAT_TC(table1, table2)
    w4 = words.reshape(NW, GROUPS, SUPER, CHUNK).astype(jnp.int32)
    out = _STACK_EMBED(w4, ta, table2)
    return out.reshape(BATCH, SEQ, DIM)


# speedup vs baseline: 1.0508x; 1.0508x over previous
import functools

import jax
import jax.numpy as jnp
from jax import lax
from jax.experimental import pallas as pl
from jax.experimental.pallas import tpu as pltpu
from jax.experimental.pallas import tpu_sc as plsc

VOCAB = 100000
DIM1 = 64
DIM2 = 128
DIM = DIM1 + DIM2
BATCH = 4096
SEQ = 200
N = BATCH * SEQ

NUM_CORES = 2
NUM_SUBCORES = 16
NW = NUM_CORES * NUM_SUBCORES
PER_W = N // NW

CHUNK = 128
STEPS = PER_W // CHUNK
SUPER = 10
GROUPS = STEPS // SUPER
LANES = 16


def _make_kernel():
    mesh = plsc.VectorSubcoreMesh(core_axis_name="c", subcore_axis_name="s")

    @functools.partial(
        pl.kernel,
        mesh=mesh,
        out_type=jax.ShapeDtypeStruct((N // CHUNK, CHUNK, DIM), jnp.float32),
        scratch_types=[
            pltpu.VMEM((SUPER, CHUNK), jnp.int32),
            pltpu.VMEM((SUPER, CHUNK), jnp.int32),
            pltpu.VMEM((CHUNK, DIM), jnp.float32),
            pltpu.VMEM((CHUNK, DIM), jnp.float32),
            pltpu.VMEM((CHUNK, DIM2), jnp.float32),
            pltpu.VMEM((CHUNK, DIM2), jnp.float32),
            pltpu.SemaphoreType.DMA,
            pltpu.SemaphoreType.DMA,
            pltpu.SemaphoreType.DMA,
            pltpu.SemaphoreType.DMA,
        ],
    )
    def stack_embed(
        words_hbm,
        ta_hbm,
        t2_hbm,
        out_hbm,
        idx_a,
        idx_b,
        comb_a,
        comb_b,
        r2_a,
        r2_b,
        sem_ga,
        sem_gb,
        sem_ia,
        sem_ib,
    ):
        wid = lax.axis_index("s") * NUM_CORES + lax.axis_index("c")
        base = wid * PER_W
        idx = (idx_a, idx_b)
        comb = (comb_a, comb_b)
        r2 = (r2_a, r2_b)
        sem_g = (sem_ga, sem_gb)
        sem_i = (sem_ia, sem_ib)

        def issue(idx_row, b):
            pltpu.async_copy(
                ta_hbm.at[idx_row], comb[b].at[:, pl.ds(0, DIM2)], sem_g[b]
            )
            pltpu.async_copy(t2_hbm.at[idx_row], r2[b], sem_g[b])

        def drain(idx_row, b):
            pltpu.make_async_copy(
                ta_hbm.at[idx_row], comb[b].at[:, pl.ds(0, DIM2)], sem_g[b]
            ).wait()
            pltpu.make_async_copy(t2_hbm.at[idx_row], r2[b], sem_g[b]).wait()

        def process(j, b, cur_row, issue_next):
            drain(cur_row, b)
            issue_next()

            def tail2(jj, c):
                for r in range(2):
                    row = jj * 2 + r
                    for k in range(DIM1 // LANES):
                        comb[b][row, pl.ds(DIM2 + k * LANES, LANES)] = r2[b][
                            row, pl.ds(DIM1 + k * LANES, LANES)
                        ]
                return c

            lax.fori_loop(0, CHUNK // 2, tail2, 0)
            pltpu.sync_copy(comb[b], out_hbm.at[wid * STEPS + j])

        def group(g, p, prefetch, t):
            prefetch()
            for c in range(SUPER):
                b = c % 2
                if c < SUPER - 1:
                    nxt = lambda c=c, b=b: issue(idx[p].at[c + 1], b ^ 1)
                elif p == 0:
                    def nxt(b=b):
                        pltpu.make_async_copy(
                            words_hbm.at[wid, 0], idx[1], sem_i[1]
                        ).wait()
                        issue(idx[1].at[0], b ^ 1)

                else:
                    def nxt(b=b):
                        @pl.when(t < GROUPS // 2 - 1)
                        def _():
                            pltpu.make_async_copy(
                                words_hbm.at[wid, 0], idx[0], sem_i[0]
                            ).wait()
                            issue(idx[0].at[0], b ^ 1)

                process(g * SUPER + c, b, idx[p].at[c], nxt)

        def body(t, carry):
            g0 = t * 2
            g1 = g0 + 1

            def pre0():
                pltpu.async_copy(words_hbm.at[wid, g1], idx[1], sem_i[1])

            def pre1():
                @pl.when(t < GROUPS // 2 - 1)
                def _():
                    pltpu.async_copy(words_hbm.at[wid, g0 + 2], idx[0], sem_i[0])

            group(g0, 0, pre0, t)
            group(g1, 1, pre1, t)
            return carry

        pltpu.sync_copy(words_hbm.at[wid, 0], idx[0])
        issue(idx[0].at[0], 0)
        lax.fori_loop(0, GROUPS // 2, body, 0)

    return stack_embed


_STACK_EMBED = _make_kernel()

_TC_BLOCK = 1000


def _concat_body(t1_ref, t2_ref, o_ref):
    o_ref[:, 0:DIM1] = t1_ref[...]
    o_ref[:, DIM1:DIM2] = t2_ref[:, 0:DIM1]


_CONCAT_TC = pl.pallas_call(
    _concat_body,
    grid=(VOCAB // _TC_BLOCK,),
    in_specs=[
        pl.BlockSpec((_TC_BLOCK, DIM1), lambda i: (i, 0)),
        pl.BlockSpec((_TC_BLOCK, DIM2), lambda i: (i, 0)),
    ],
    out_specs=pl.BlockSpec((_TC_BLOCK, DIM2), lambda i: (i, 0)),
    out_shape=jax.ShapeDtypeStruct((VOCAB, DIM2), jnp.float32),
)


def kernel(words, table1, table2):
    ta = jnp.concatenate([table1, table2[:, :DIM1]], axis=1)
    w4 = words.reshape(NW, GROUPS, SUPER, CHUNK).astype(jnp.int32)
    out = _STACK_EMBED(w4, ta, table2)
    return out.reshape(BATCH, SEQ, DIM)
